# baseline (speedup 1.0000x reference)
import jax
import jax.numpy as jnp
from jax.experimental import pallas as pl
from jax.experimental.pallas import tpu as pltpu

_NF = 128


def _tc_body(user_p, att_p, item_p, uf_hbm, tf_hbm, itf_ref,
             tb_hbm, dub_hbm, dib_hbm, out_ref,
             trow_v, u_sm, btb_v, bdu_v, bdi_v,
             semt0, semt1, semu, sem0, sem1, sem2):
    u0 = user_p[0]
    a0 = att_p[0]
    i0 = item_p[0]
    ir = i0 % 8

    half = _NF * _NF // 2
    ct0 = pltpu.make_async_copy(
        tf_hbm.at[pl.ds(a0, 1), pl.ds(0, half)], trow_v.at[:, pl.ds(0, half)],
        semt0)
    ct1 = pltpu.make_async_copy(
        tf_hbm.at[pl.ds(a0, 1), pl.ds(half, half)],
        trow_v.at[:, pl.ds(half, half)], semt1)
    cu = pltpu.make_async_copy(uf_hbm.at[pl.ds(u0, 1), :], u_sm, semu)
    ab = pl.multiple_of((a0 // 128) * 128, 128)
    ub = pl.multiple_of((u0 // 128) * 128, 128)
    ib = pl.multiple_of((i0 // 128) * 128, 128)
    cb0 = pltpu.make_async_copy(tb_hbm.at[pl.ds(ab, 128)], btb_v, sem0)
    cb1 = pltpu.make_async_copy(dub_hbm.at[pl.ds(ub, 128)], bdu_v, sem1)
    cb2 = pltpu.make_async_copy(dib_hbm.at[pl.ds(ib, 128)], bdi_v, sem2)
    ct0.start()
    ct1.start()
    cu.start()
    cb0.start()
    cb1.start()
    cb2.start()

    rows = jax.lax.broadcasted_iota(jnp.int32, (8, _NF), 0)
    i_sel = jnp.sum(jnp.where(rows == ir, itf_ref[...], 0.0), axis=0,
                    keepdims=True)

    cu.wait()
    ct0.wait()
    y = jnp.zeros((1, _NF), jnp.float32)
    for a in range(_NF // 2):
        y = y + u_sm[0, a] * trow_v[:, pl.ds(a * _NF, _NF)]
    ct1.wait()
    for a in range(_NF // 2, _NF):
        y = y + u_sm[0, a] * trow_v[:, pl.ds(a * _NF, _NF)]
    pred = jnp.sum(y * i_sel)

    cb0.wait()
    cb1.wait()
    cb2.wait()
    pred = pred + btb_v[a0 % 128] + bdu_v[u0 % 128] + bdi_v[i0 % 128]
    out_ref[0, 0] = 1.0 / (1.0 + jnp.exp(-pred))


def _tc_call(u32, a32, i32, uf, tf, itf, tb, dub, dib):
    grid_spec = pltpu.PrefetchScalarGridSpec(
        num_scalar_prefetch=3,
        grid=(1,),
        in_specs=[
            pl.BlockSpec(memory_space=pl.ANY),
            pl.BlockSpec(memory_space=pl.ANY),
            pl.BlockSpec((8, _NF), lambda g, u, a, i: (i[0] // 8, 0)),
            pl.BlockSpec(memory_space=pl.ANY),
            pl.BlockSpec(memory_space=pl.ANY),
            pl.BlockSpec(memory_space=pl.ANY),
        ],
        out_specs=pl.BlockSpec((1, 1), lambda g, u, a, i: (0, 0),
                               memory_space=pltpu.SMEM),
        scratch_shapes=[
            pltpu.VMEM((1, _NF * _NF), jnp.float32),
            pltpu.SMEM((1, _NF), jnp.float32),
            pltpu.SMEM((128,), jnp.float32),
            pltpu.SMEM((128,), jnp.float32),
            pltpu.SMEM((128,), jnp.float32),
            pltpu.SemaphoreType.DMA,
            pltpu.SemaphoreType.DMA,
            pltpu.SemaphoreType.DMA,
            pltpu.SemaphoreType.DMA,
            pltpu.SemaphoreType.DMA,
            pltpu.SemaphoreType.DMA,
        ],
    )
    out = pl.pallas_call(
        _tc_body, grid_spec=grid_spec,
        out_shape=jax.ShapeDtypeStruct((1, 1), jnp.float32),
    )(u32, a32, i32, uf, tf, itf, tb, dub, dib)
    return out.reshape(1)


def kernel(user, attempt, item, view, user_factors, time_factors, item_factors,
           stress_item_factor, time_biases, stress_user_biases,
           stress_item_biases, rate_user_biases, rate_item_biases,
           done_user_biases, done_item_biases):
    del view, stress_item_factor, stress_user_biases, stress_item_biases
    del rate_user_biases, rate_item_biases
    return _tc_call(user.astype(jnp.int32), attempt.astype(jnp.int32),
                    item.astype(jnp.int32), user_factors, time_factors,
                    item_factors, time_biases.reshape(-1),
                    done_user_biases.reshape(-1), done_item_biases.reshape(-1))


# speedup vs baseline: 1.0086x; 1.0086x over previous
import jax
import jax.numpy as jnp
from jax.experimental import pallas as pl
from jax.experimental.pallas import tpu as pltpu

_NF = 128


def _tc_body(user_p, att_p, item_p, uf_hbm, tf_hbm, itf_ref,
             tb_hbm, dub_hbm, dib_hbm, out_ref,
             trow_v, u_sm, btb_v, bdu_v, bdi_v,
             semt0, semt1, semu, sem0, sem1, sem2):
    u0 = user_p[0]
    a0 = att_p[0]
    i0 = item_p[0]
    ir = i0 % 8

    half = _NF * _NF // 2
    ct0 = pltpu.make_async_copy(
        tf_hbm.at[pl.ds(a0, 1), pl.ds(0, half)], trow_v.at[:, pl.ds(0, half)],
        semt0)
    ct1 = pltpu.make_async_copy(
        tf_hbm.at[pl.ds(a0, 1), pl.ds(half, half)],
        trow_v.at[:, pl.ds(half, half)], semt1)
    cu = pltpu.make_async_copy(uf_hbm.at[pl.ds(u0, 1), :], u_sm, semu)
    ab = pl.multiple_of((a0 // 128) * 128, 128)
    ub = pl.multiple_of((u0 // 128) * 128, 128)
    ib = pl.multiple_of((i0 // 128) * 128, 128)
    cb0 = pltpu.make_async_copy(tb_hbm.at[pl.ds(ab, 128)], btb_v, sem0)
    cb1 = pltpu.make_async_copy(dub_hbm.at[pl.ds(ub, 128)], bdu_v, sem1)
    cb2 = pltpu.make_async_copy(dib_hbm.at[pl.ds(ib, 128)], bdi_v, sem2)
    ct0.start()
    ct1.start()
    cu.start()
    cb0.start()
    cb1.start()
    cb2.start()

    rows = jax.lax.broadcasted_iota(jnp.int32, (8, _NF), 0)
    i_sel = jnp.sum(jnp.where(rows == ir, itf_ref[...], 0.0), axis=0,
                    keepdims=True)

    cu.wait()
    ct0.wait()
    accs = [jnp.zeros((1, _NF), jnp.float32) for _ in range(8)]
    for a in range(_NF // 2):
        accs[a % 8] = accs[a % 8] + u_sm[0, a] * trow_v[:, pl.ds(a * _NF, _NF)]
    ct1.wait()
    for a in range(_NF // 2, _NF):
        accs[a % 8] = accs[a % 8] + u_sm[0, a] * trow_v[:, pl.ds(a * _NF, _NF)]
    y = (((accs[0] + accs[1]) + (accs[2] + accs[3]))
         + ((accs[4] + accs[5]) + (accs[6] + accs[7])))
    pred = jnp.sum(y * i_sel)

    cb0.wait()
    cb1.wait()
    cb2.wait()
    pred = pred + btb_v[a0 % 128] + bdu_v[u0 % 128] + bdi_v[i0 % 128]
    out_ref[0, 0] = 1.0 / (1.0 + jnp.exp(-pred))


def _tc_call(u32, a32, i32, uf, tf, itf, tb, dub, dib):
    grid_spec = pltpu.PrefetchScalarGridSpec(
        num_scalar_prefetch=3,
        grid=(1,),
        in_specs=[
            pl.BlockSpec(memory_space=pl.ANY),
            pl.BlockSpec(memory_space=pl.ANY),
            pl.BlockSpec((8, _NF), lambda g, u, a, i: (i[0] // 8, 0)),
            pl.BlockSpec(memory_space=pl.ANY),
            pl.BlockSpec(memory_space=pl.ANY),
            pl.BlockSpec(memory_space=pl.ANY),
        ],
        out_specs=pl.BlockSpec((1, 1), lambda g, u, a, i: (0, 0),
                               memory_space=pltpu.SMEM),
        scratch_shapes=[
            pltpu.VMEM((1, _NF * _NF), jnp.float32),
            pltpu.SMEM((1, _NF), jnp.float32),
            pltpu.SMEM((128,), jnp.float32),
            pltpu.SMEM((128,), jnp.float32),
            pltpu.SMEM((128,), jnp.float32),
            pltpu.SemaphoreType.DMA,
            pltpu.SemaphoreType.DMA,
            pltpu.SemaphoreType.DMA,
            pltpu.SemaphoreType.DMA,
            pltpu.SemaphoreType.DMA,
            pltpu.SemaphoreType.DMA,
        ],
    )
    out = pl.pallas_call(
        _tc_body, grid_spec=grid_spec,
        out_shape=jax.ShapeDtypeStruct((1, 1), jnp.float32),
    )(u32, a32, i32, uf, tf, itf, tb, dub, dib)
    return out.reshape(1)


def kernel(user, attempt, item, view, user_factors, time_factors, item_factors,
           stress_item_factor, time_biases, stress_user_biases,
           stress_item_biases, rate_user_biases, rate_item_biases,
           done_user_biases, done_item_biases):
    del view, stress_item_factor, stress_user_biases, stress_item_biases
    del rate_user_biases, rate_item_biases
    return _tc_call(user.astype(jnp.int32), attempt.astype(jnp.int32),
                    item.astype(jnp.int32), user_factors, time_factors,
                    item_factors, time_biases.reshape(-1),
                    done_user_biases.reshape(-1), done_item_biases.reshape(-1))


# speedup vs baseline: 1.0207x; 1.0120x over previous
import jax
import jax.numpy as jnp
from jax.experimental import pallas as pl
from jax.experimental.pallas import tpu as pltpu

_NF = 128
_HI = jax.lax.Precision.HIGHEST


def _sel128(vec_ref, idx):
    v = vec_ref[...].reshape(1, _NF)
    lanes = jax.lax.broadcasted_iota(jnp.int32, (1, _NF), 1)
    return jnp.sum(jnp.where(lanes == idx % _NF, v, 0.0))


def _tc_body(user_p, att_p, item_p, uf_ref, tf_ref, itf_ref,
             tb_ref, dub_ref, dib_ref, out_ref):
    u0 = user_p[0]
    a0 = att_p[0]
    i0 = item_p[0]
    ur = u0 % 8
    ar = a0 % 8
    ir = i0 % 8

    rows = jax.lax.broadcasted_iota(jnp.int32, (8, _NF), 0)
    u_sel = jnp.sum(jnp.where(rows == ur, uf_ref[...], 0.0), axis=0,
                    keepdims=True)
    i_sel = jnp.sum(jnp.where(rows == ir, itf_ref[...], 0.0), axis=0,
                    keepdims=True)

    ident = (jax.lax.broadcasted_iota(jnp.int32, (_NF, _NF), 0)
             == jax.lax.broadcasted_iota(jnp.int32, (_NF, _NF), 1)
             ).astype(jnp.float32)
    u_col = jax.lax.dot_general(ident, u_sel, (((1,), (1,)), ((), ())),
                                precision=_HI,
                                preferred_element_type=jnp.float32)
    outer = jax.lax.dot_general(u_col, i_sel, (((1,), (0,)), ((), ())),
                                precision=_HI,
                                preferred_element_type=jnp.float32)

    s8 = jnp.zeros((8, _NF), jnp.float32)
    for a in range(_NF):
        s8 = s8 + tf_ref[:, pl.ds(a * _NF, _NF)] * outer[a:a + 1, :]
    pred = jnp.sum(jnp.where(rows == ar, s8, 0.0))

    pred = (pred + _sel128(tb_ref, a0) + _sel128(dub_ref, u0)
            + _sel128(dib_ref, i0))
    out_ref[0, 0] = 1.0 / (1.0 + jnp.exp(-pred))


def _tc_call(u32, a32, i32, uf, tf, itf, tb, dub, dib):
    grid_spec = pltpu.PrefetchScalarGridSpec(
        num_scalar_prefetch=3,
        grid=(1,),
        in_specs=[
            pl.BlockSpec((8, _NF), lambda g, u, a, i: (u[0] // 8, 0)),
            pl.BlockSpec((8, 16384), lambda g, u, a, i: (a[0] // 8, 0)),
            pl.BlockSpec((8, _NF), lambda g, u, a, i: (i[0] // 8, 0)),
            pl.BlockSpec((_NF,), lambda g, u, a, i: (a[0] // _NF,)),
            pl.BlockSpec((_NF,), lambda g, u, a, i: (u[0] // _NF,)),
            pl.BlockSpec((_NF,), lambda g, u, a, i: (i[0] // _NF,)),
        ],
        out_specs=pl.BlockSpec((1, 1), lambda g, u, a, i: (0, 0),
                               memory_space=pltpu.SMEM),
    )
    out = pl.pallas_call(
        _tc_body, grid_spec=grid_spec,
        out_shape=jax.ShapeDtypeStruct((1, 1), jnp.float32),
    )(u32, a32, i32, uf, tf, itf, tb, dub, dib)
    return out.reshape(1)


def kernel(user, attempt, item, view, user_factors, time_factors, item_factors,
           stress_item_factor, time_biases, stress_user_biases,
           stress_item_biases, rate_user_biases, rate_item_biases,
           done_user_biases, done_item_biases):
    del view, stress_item_factor, stress_user_biases, stress_item_biases
    del rate_user_biases, rate_item_biases
    return _tc_call(user.astype(jnp.int32), attempt.astype(jnp.int32),
                    item.astype(jnp.int32), user_factors, time_factors,
                    item_factors, time_biases.reshape(-1),
                    done_user_biases.reshape(-1), done_item_biases.reshape(-1))
